# Initial kernel scaffold; baseline (speedup 1.0000x reference)
import jax
import jax.numpy as jnp
from jax.experimental import pallas as pl


def kernel(x, weight):
    raise NotImplementedError("write your pallas kernel here")


# speedup vs baseline: 1.3293x; 1.3293x over previous
import functools

import jax
import jax.numpy as jnp
from jax.experimental import pallas as pl
from jax.experimental.pallas import tpu as pltpu


def _conv_relu_kernel(x_ref, w_ref, o_ref, *, kh, kw, W, Q, cin):
    xb = x_ref[0].astype(jnp.bfloat16)
    xb = jnp.concatenate(
        [xb, jnp.zeros((cin, 128), jnp.bfloat16)], axis=1)

    taps = []
    for ki in range(kh):
        for kj in range(kw):
            s = ki * W + kj
            taps.append(xb[:, s:s + Q])
    patch = jnp.concatenate(taps, axis=0)

    acc = jnp.dot(w_ref[...], patch, preferred_element_type=jnp.float32)
    o_ref[0] = jnp.maximum(acc, 0.0).astype(jnp.bfloat16)


@jax.jit
def _forward(x, weight):
    B, C_in, H, W = x.shape
    C_out, _, kh, kw = weight.shape
    Ho = H - kh + 1
    Wo = W - kw + 1
    Q = Ho * W
    HW = H * W

    w = jnp.transpose(weight.astype(jnp.bfloat16), (0, 2, 3, 1))
    w = w.reshape(C_out, kh * kw * C_in)

    x_flat = x.reshape(B, C_in, HW)

    body = functools.partial(_conv_relu_kernel, kh=kh, kw=kw, W=W, Q=Q, cin=C_in)

    flops = 2 * B * C_out * (kh * kw * C_in) * Q
    bytes_accessed = x_flat.size * 4 + w.size * 2 + B * C_out * Q * 2

    out = pl.pallas_call(
        body,
        out_shape=jax.ShapeDtypeStruct((B, C_out, Q), jnp.bfloat16),
        grid_spec=pltpu.PrefetchScalarGridSpec(
            num_scalar_prefetch=0,
            grid=(B,),
            in_specs=[
                pl.BlockSpec((1, C_in, HW), lambda b: (b, 0, 0)),
                pl.BlockSpec((C_out, kh * kw * C_in), lambda b: (0, 0)),
            ],
            out_specs=pl.BlockSpec((1, C_out, Q), lambda b: (b, 0, 0)),
        ),
        compiler_params=pltpu.CompilerParams(
            dimension_semantics=("parallel",),
            vmem_limit_bytes=64 * 1024 * 1024),
        cost_estimate=pl.CostEstimate(flops=flops, transcendentals=0,
                                      bytes_accessed=bytes_accessed),
    )(x_flat, w)

    y = out.reshape(B, C_out, Ho, W)[:, :, :, :Wo].astype(jnp.float32)
    return y


def kernel(x, weight):
    return _forward(x, weight)
